# Initial kernel scaffold; baseline (speedup 1.0000x reference)
import jax
import jax.numpy as jnp
from jax.experimental import pallas as pl


def kernel(node_features, edge_list):
    raise NotImplementedError("write your pallas kernel here")


# speedup vs baseline: 4.5835x; 4.5835x over previous
import functools

import jax
import jax.numpy as jnp
from jax import lax
from jax.experimental import pallas as pl
from jax.experimental.pallas import tpu as pltpu
from jax.experimental.pallas import tpu_sc as plsc

N_NODES = 10000
N_EDGES = 320000
D_FEAT = 128

B = N_EDGES * 2
NW = 32
PER_W = B // NW
CHUNK = 80
N_CHUNKS = PER_W // CHUNK


def _sc_gather(table, idx2d):
    mesh = plsc.VectorSubcoreMesh(core_axis_name="c", subcore_axis_name="s")

    @functools.partial(
        pl.kernel,
        mesh=mesh,
        out_type=jax.ShapeDtypeStruct((B, D_FEAT), jnp.float32),
        scratch_types=[
            pltpu.VMEM((N_CHUNKS, CHUNK), jnp.int32),
            pltpu.VMEM((CHUNK, D_FEAT), jnp.float32),
            pltpu.SemaphoreType.DMA,
        ],
    )
    def k(table_hbm, idx_hbm, out_hbm, idx_v, rows_v, sem):
        wid = lax.axis_index("s") * 2 + lax.axis_index("c")
        pltpu.sync_copy(idx_hbm.at[wid], idx_v)

        def body(j, carry):
            pltpu.async_copy(table_hbm.at[idx_v.at[j]], rows_v, sem).wait()
            pltpu.sync_copy(
                rows_v, out_hbm.at[pl.ds(wid * PER_W + j * CHUNK, CHUNK)]
            )
            return carry

        lax.fori_loop(0, N_CHUNKS, body, 0)

    return k(table, idx2d)


def kernel(node_features, edge_list):
    idx2d = edge_list.astype(jnp.int32).reshape(NW, N_CHUNKS, CHUNK)
    out = _sc_gather(node_features, idx2d)
    return out.reshape(N_EDGES, 2, D_FEAT)


# speedup vs baseline: 6.4269x; 1.4022x over previous
import functools

import jax
import jax.numpy as jnp
from jax import lax
from jax.experimental import pallas as pl
from jax.experimental.pallas import tpu as pltpu
from jax.experimental.pallas import tpu_sc as plsc

N_NODES = 10000
N_EDGES = 320000
D_FEAT = 128

B = N_EDGES * 2
NW = 32
PER_W = B // NW
CHUNK = 80
N_CHUNKS = PER_W // CHUNK
NBUF = 5
N_GROUPS = N_CHUNKS // NBUF


def _sc_gather(table, idx3d):
    mesh = plsc.VectorSubcoreMesh(core_axis_name="c", subcore_axis_name="s")

    @functools.partial(
        pl.kernel,
        mesh=mesh,
        out_type=jax.ShapeDtypeStruct((B, D_FEAT), jnp.float32),
        scratch_types=[
            pltpu.VMEM((N_CHUNKS, CHUNK), jnp.int32),
            pltpu.VMEM((NBUF, CHUNK, D_FEAT), jnp.float32),
            pltpu.SemaphoreType.DMA,
            pltpu.SemaphoreType.DMA,
        ],
    )
    def k(table_hbm, idx_hbm, out_hbm, idx_v, rows_v, gsem, wsem):
        wid = lax.axis_index("s") * 2 + lax.axis_index("c")
        pltpu.sync_copy(idx_hbm.at[wid], idx_v)

        for b in range(NBUF):
            pltpu.async_copy(table_hbm.at[idx_v.at[b]], rows_v.at[b], gsem)

        def body(g, carry):
            base = g * NBUF
            for b in range(NBUF):
                j = base + b
                pltpu.make_async_copy(
                    table_hbm.at[idx_v.at[j]], rows_v.at[b], gsem
                ).wait()
                pltpu.async_copy(
                    rows_v.at[b],
                    out_hbm.at[pl.ds(wid * PER_W + j * CHUNK, CHUNK)],
                    wsem,
                )
            for b in range(NBUF):
                j = base + b
                pltpu.make_async_copy(
                    rows_v.at[b],
                    out_hbm.at[pl.ds(wid * PER_W + j * CHUNK, CHUNK)],
                    wsem,
                ).wait()

                @pl.when(g < N_GROUPS - 1)
                def _():
                    pltpu.async_copy(
                        table_hbm.at[idx_v.at[j + NBUF]], rows_v.at[b], gsem
                    )

            return carry

        lax.fori_loop(0, N_GROUPS, body, 0)

    return k(table, idx3d)


def kernel(node_features, edge_list):
    idx3d = edge_list.astype(jnp.int32).reshape(NW, N_CHUNKS, CHUNK)
    out = _sc_gather(node_features, idx3d)
    return out.reshape(N_EDGES, 2, D_FEAT)


# speedup vs baseline: 9.3159x; 1.4495x over previous
import functools

import jax
import jax.numpy as jnp
from jax import lax
from jax.experimental import pallas as pl
from jax.experimental.pallas import tpu as pltpu
from jax.experimental.pallas import tpu_sc as plsc

N_NODES = 10000
N_EDGES = 320000
D_FEAT = 128

B = N_EDGES * 2
NW = 32
PER_W = B // NW
CHUNK = 80
N_CHUNKS = PER_W // CHUNK
NBUF = 2
N_PARTS = 5
PART = N_CHUNKS // N_PARTS
N_GROUPS = PART // NBUF
STAGE_ROWS = 1000


def _sc_gather(table, idx4d):
    mesh = plsc.VectorSubcoreMesh(core_axis_name="c", subcore_axis_name="s")

    @functools.partial(
        pl.kernel,
        mesh=mesh,
        out_type=jax.ShapeDtypeStruct((B, D_FEAT), jnp.float32),
        scratch_types=[
            pltpu.VMEM_SHARED((N_NODES, D_FEAT), jnp.float32),
            pltpu.VMEM((PART, CHUNK), jnp.int32),
            pltpu.VMEM((NBUF, CHUNK, D_FEAT), jnp.float32),
            pltpu.SemaphoreType.DMA,
            pltpu.SemaphoreType.DMA,
        ],
    )
    def k(table_hbm, idx_hbm, out_hbm, table_s, idx_v, rows_v, gsem, wsem):
        cid = lax.axis_index("c")
        sid = lax.axis_index("s")
        wid = sid * 2 + cid

        @pl.when(sid < N_NODES // STAGE_ROWS)
        def _():
            pltpu.sync_copy(
                table_hbm.at[pl.ds(sid * STAGE_ROWS, STAGE_ROWS)],
                table_s.at[pl.ds(sid * STAGE_ROWS, STAGE_ROWS)],
            )

        pltpu.sync_copy(idx_hbm.at[wid, 0], idx_v)
        plsc.subcore_barrier()

        def run_part(p):
            out0 = wid * PER_W + p * PART * CHUNK

            for b in range(NBUF):
                pltpu.async_copy(table_s.at[idx_v.at[b]], rows_v.at[b], gsem)

            def body(g, carry):
                base = g * NBUF
                for b in range(NBUF):
                    j = base + b
                    pltpu.make_async_copy(
                        table_s.at[idx_v.at[j]], rows_v.at[b], gsem
                    ).wait()
                    pltpu.async_copy(
                        rows_v.at[b],
                        out_hbm.at[pl.ds(out0 + j * CHUNK, CHUNK)],
                        wsem,
                    )
                for b in range(NBUF):
                    j = base + b
                    pltpu.make_async_copy(
                        rows_v.at[b],
                        out_hbm.at[pl.ds(out0 + j * CHUNK, CHUNK)],
                        wsem,
                    ).wait()

                    @pl.when(g < N_GROUPS - 1)
                    def _():
                        pltpu.async_copy(
                            table_s.at[idx_v.at[j + NBUF]], rows_v.at[b], gsem
                        )

                return carry

            lax.fori_loop(0, N_GROUPS, body, 0)

        run_part(0)
        for p in range(1, N_PARTS):
            pltpu.sync_copy(idx_hbm.at[wid, p], idx_v)
            run_part(p)

    return k(table, idx4d)


def kernel(node_features, edge_list):
    idx4d = edge_list.astype(jnp.int32).reshape(NW, N_PARTS, PART, CHUNK)
    out = _sc_gather(node_features, idx4d)
    return out.reshape(N_EDGES, 2, D_FEAT)


# speedup vs baseline: 9.3213x; 1.0006x over previous
import functools

import jax
import jax.numpy as jnp
from jax import lax
from jax.experimental import pallas as pl
from jax.experimental.pallas import tpu as pltpu
from jax.experimental.pallas import tpu_sc as plsc

N_NODES = 10000
N_EDGES = 320000
D_FEAT = 128

B = N_EDGES * 2
NW = 32
PER_W = B // NW
CHUNK = 40
N_CHUNKS = PER_W // CHUNK
NBUF = 5
N_PARTS = 10
PART = N_CHUNKS // N_PARTS
N_GROUPS = PART // NBUF
STAGE_ROWS = 1000


def _sc_gather(table, idx4d):
    mesh = plsc.VectorSubcoreMesh(core_axis_name="c", subcore_axis_name="s")

    @functools.partial(
        pl.kernel,
        mesh=mesh,
        out_type=jax.ShapeDtypeStruct((B, D_FEAT), jnp.float32),
        scratch_types=[
            pltpu.VMEM_SHARED((N_NODES, D_FEAT), jnp.float32),
            pltpu.VMEM((PART, CHUNK), jnp.int32),
            pltpu.VMEM((NBUF, CHUNK, D_FEAT), jnp.float32),
            pltpu.SemaphoreType.DMA,
            pltpu.SemaphoreType.DMA,
        ],
    )
    def k(table_hbm, idx_hbm, out_hbm, table_s, idx_v, rows_v, gsem, wsem):
        cid = lax.axis_index("c")
        sid = lax.axis_index("s")
        wid = sid * 2 + cid

        @pl.when(sid < N_NODES // STAGE_ROWS)
        def _():
            pltpu.sync_copy(
                table_hbm.at[pl.ds(sid * STAGE_ROWS, STAGE_ROWS)],
                table_s.at[pl.ds(sid * STAGE_ROWS, STAGE_ROWS)],
            )

        pltpu.sync_copy(idx_hbm.at[wid, 0], idx_v)
        plsc.subcore_barrier()

        def run_part(p):
            out0 = wid * PER_W + p * PART * CHUNK

            for b in range(NBUF):
                pltpu.async_copy(table_s.at[idx_v.at[b]], rows_v.at[b], gsem)

            def body(g, carry):
                base = g * NBUF
                for b in range(NBUF):
                    j = base + b
                    pltpu.make_async_copy(
                        table_s.at[idx_v.at[j]], rows_v.at[b], gsem
                    ).wait()
                    pltpu.async_copy(
                        rows_v.at[b],
                        out_hbm.at[pl.ds(out0 + j * CHUNK, CHUNK)],
                        wsem,
                    )
                for b in range(NBUF):
                    j = base + b
                    pltpu.make_async_copy(
                        rows_v.at[b],
                        out_hbm.at[pl.ds(out0 + j * CHUNK, CHUNK)],
                        wsem,
                    ).wait()

                    @pl.when(g < N_GROUPS - 1)
                    def _():
                        pltpu.async_copy(
                            table_s.at[idx_v.at[j + NBUF]], rows_v.at[b], gsem
                        )

                return carry

            lax.fori_loop(0, N_GROUPS, body, 0)

        run_part(0)
        for p in range(1, N_PARTS):
            pltpu.sync_copy(idx_hbm.at[wid, p], idx_v)
            run_part(p)

    return k(table, idx4d)


def kernel(node_features, edge_list):
    idx4d = edge_list.astype(jnp.int32).reshape(NW, N_PARTS, PART, CHUNK)
    out = _sc_gather(node_features, idx4d)
    return out.reshape(N_EDGES, 2, D_FEAT)
